# baseline (speedup 1.0000x reference)
import functools

import jax
import jax.numpy as jnp
from jax import lax
from jax.experimental import pallas as pl
from jax.experimental.pallas import tpu as pltpu
from jax.experimental.pallas import tpu_sc as plsc

N = 10000
E = 320000
C = 128
A = 16
R = 8
H = 8
AVG_NEIGH = 32.0

NC = 2
NS = 16
L = 16
CP = C // 2

NB = 1000
EB = 6400
K = 80
EPW = E // (NC * NS)
NBLK = EPW // K
RPT = 624
TAIL = N - NS * RPT


def _pack_bf16(v):
    return pltpu.pack_elementwise([v[:, :CP], v[:, CP:]],
                                  packed_dtype=jnp.bfloat16)


def _xl_kernel(x_ref, wl_ref, xl_ref):
    xl_ref[...] = jnp.dot(x_ref[...], wl_ref[...] * (1.0 / jnp.sqrt(float(C))),
                          preferred_element_type=jnp.float32)


def _sc_term_kernel(x_ref, na_ref, wsc_ref, sc_ref):
    xb = x_ref[...]
    acc = jnp.zeros((NB, C), dtype=jnp.float32)
    for a in range(A):
        acc = acc + jnp.dot(xb * na_ref[:, a][:, None], wsc_ref[:, a, :],
                            preferred_element_type=jnp.float32)
    sc_ref[...] = acc * (1.0 / jnp.sqrt(float(C * A)))


def _split_idx_kernel(ei_ref, src_ref, dst_ref):
    src_ref[...] = ei_ref[0]
    dst_ref[...] = ei_ref[1]


def _edge_kernel(radT_ref, sphT_ref, wf1_ref, wf2_ref, w2p_ref):
    preT = lax.dot_general(wf1_ref[...] * (1.0 / jnp.sqrt(float(R))),
                           radT_ref[...],
                           (((0,), (0,)), ((), ())),
                           preferred_element_type=jnp.float32)
    hT = jax.nn.softplus(preT) - jnp.log(2.0)
    w = lax.dot_general(hT,
                        wf2_ref[...] * (1.0 / jnp.sqrt(float(H) * AVG_NEIGH)),
                        (((0,), (0,)), ((), ())),
                        preferred_element_type=jnp.float32)
    sph_col = jnp.transpose(sphT_ref[...], (1, 0))
    w2p_ref[...] = _pack_bf16(w * sph_col)


def _combine_kernel(acc_ref, sc_ref, out_ref):
    out_ref[...] = acc_ref[0] + acc_ref[1] + sc_ref[...]


HI_MASK = -65536


def _sc_kernel(xl_hbm, w2p_hbm, src_hbm, dst_hbm, out_hbm,
               acc, srcb0, srcb1, srcb2, dstb0, dstb1, dstb2, rows0, rows1,
               w2b0, w2b1,
               sg0, sg1, sw0, sw1, si0, si1, si2, sd0, sd1, sd2, ss0, ss1):
    cid = lax.axis_index("c")
    sid = lax.axis_index("s")
    rows = (rows0, rows1)
    w2b = (w2b0, w2b1)
    srcb = (srcb0, srcb1, srcb2)
    dstb = (dstb0, dstb1, dstb2)
    sg = (sg0, sg1)
    sw = (sw0, sw1)
    si = (si0, si1, si2)
    sd = (sd0, sd1, sd2)
    ss = (ss0, ss1)

    def zrow(i, _):
        for k in range(C // L):
            rows0[i, pl.ds(k * L, L)] = jnp.zeros((L,), jnp.float32)
        return 0
    lax.fori_loop(0, K, zrow, 0)
    r0 = sid * RPT
    for j in range(RPT // K):
        pltpu.sync_copy(rows0, acc.at[pl.ds(r0 + j * K, K)])
    rem = RPT - (RPT // K) * K
    if rem:
        pltpu.sync_copy(rows0.at[pl.ds(0, rem)],
                        acc.at[pl.ds(r0 + (RPT // K) * K, rem)])

    @pl.when(sid == NS - 1)
    def _():
        pltpu.sync_copy(rows0.at[pl.ds(0, TAIL)], acc.at[pl.ds(NS * RPT, TAIL)])
    plsc.subcore_barrier()

    base_e = (cid * NS + sid) * EPW

    def issue_idx(t, b):
        pltpu.async_copy(src_hbm.at[pl.ds(base_e + b * K, K)], srcb[t], si[t])
        pltpu.async_copy(dst_hbm.at[pl.ds(base_e + b * K, K)], dstb[t], sd[t])

    def wait_idx(t, b):
        pltpu.make_async_copy(src_hbm.at[pl.ds(base_e + b * K, K)],
                              srcb[t], si[t]).wait()
        pltpu.make_async_copy(dst_hbm.at[pl.ds(base_e + b * K, K)],
                              dstb[t], sd[t]).wait()

    def issue_gw(s, t, b):
        pltpu.async_copy(w2p_hbm.at[pl.ds(base_e + b * K, K)], w2b[s], sw[s])
        pltpu.async_copy(xl_hbm.at[srcb[t]], rows[s], sg[s])

    def step(j, b, nxt_gw, nxt_idx, wait_sc):
        s, t = j % 2, j % 3
        if nxt_idx:
            issue_idx((j + 2) % 3, b + 2)
        pltpu.make_async_copy(w2p_hbm.at[pl.ds(base_e + b * K, K)],
                              w2b[s], sw[s]).wait()
        pltpu.make_async_copy(xl_hbm.at[srcb[t]], rows[s], sg[s]).wait()
        if nxt_gw:
            s1, t1 = (j + 1) % 2, (j + 1) % 3
            if wait_sc:
                pltpu.make_async_copy(
                    rows[s1], acc.at[dstb[(j + 2) % 3]], ss[s1]).wait()
            wait_idx(t1, b + 1)
            issue_gw(s1, t1, b + 1)

        def mul(i, _):
            for g in range(C // 32):
                wv = w2b[s][i, pl.ds(g * L, L)]
                w_lo = lax.bitcast_convert_type(wv << 16, jnp.float32)
                w_hi = lax.bitcast_convert_type(wv & HI_MASK, jnp.float32)
                slo = pl.ds(g * L, L)
                shi = pl.ds(CP + g * L, L)
                rows[s][i, slo] = rows[s][i, slo] * w_lo
                rows[s][i, shi] = rows[s][i, shi] * w_hi
            return 0
        lax.fori_loop(0, K, mul, 0)
        pltpu.async_copy(rows[s], acc.at[dstb[t]], ss[s], add=True)

    issue_idx(0, 0)
    issue_idx(1, 1)
    wait_idx(0, 0)
    issue_gw(0, 0, 0)

    for b in range(6):
        step(b, b, True, True, b >= 1)

    NMAIN = 6 + (NBLK - 5 - 6) // 6 * 6

    def body(i, _):
        b0 = 6 + 6 * i
        for j in range(6):
            step(j, b0 + j, True, True, True)
        return 0
    lax.fori_loop(0, (NMAIN - 6) // 6, body, 0)
    for b in range(NMAIN, NBLK):
        step(b % 6, b, b + 1 < NBLK, b + 2 < NBLK, b + 1 < NBLK)
    jl, jl1 = (NBLK - 1) % 6, (NBLK - 2) % 6
    pltpu.make_async_copy(rows[jl % 2], acc.at[dstb[jl % 3]],
                          ss[jl % 2]).wait()
    pltpu.make_async_copy(rows[jl1 % 2], acc.at[dstb[jl1 % 3]],
                          ss[jl1 % 2]).wait()
    plsc.subcore_barrier()

    pltpu.sync_copy(acc.at[pl.ds(r0, RPT)], out_hbm.at[cid, pl.ds(r0, RPT)])

    @pl.when(sid == NS - 1)
    def _():
        pltpu.sync_copy(acc.at[pl.ds(NS * RPT, TAIL)],
                        out_hbm.at[cid, pl.ds(NS * RPT, TAIL)])


def kernel(x, node_attrs, edge_radial, edge_spherical, edge_index,
           W_lin, W_fc1, W_fc2, W_sc):
    xl = pl.pallas_call(
        _xl_kernel,
        grid=(N // NB,),
        in_specs=[
            pl.BlockSpec((NB, C), lambda i: (i, 0)),
            pl.BlockSpec((C, C), lambda i: (0, 0)),
        ],
        out_specs=pl.BlockSpec((NB, C), lambda i: (i, 0)),
        out_shape=jax.ShapeDtypeStruct((N, C), jnp.float32),
    )(x, W_lin)

    sc = pl.pallas_call(
        _sc_term_kernel,
        grid=(N // NB,),
        in_specs=[
            pl.BlockSpec((NB, C), lambda i: (i, 0)),
            pl.BlockSpec((NB, A), lambda i: (i, 0)),
            pl.BlockSpec((C, A, C), lambda i: (0, 0, 0)),
        ],
        out_specs=pl.BlockSpec((NB, C), lambda i: (i, 0)),
        out_shape=jax.ShapeDtypeStruct((N, C), jnp.float32),
    )(x, node_attrs, W_sc)

    w2p = pl.pallas_call(
        _edge_kernel,
        grid=(E // EB,),
        in_specs=[
            pl.BlockSpec((R, EB), lambda i: (0, i)),
            pl.BlockSpec((1, EB), lambda i: (0, i)),
            pl.BlockSpec((R, H), lambda i: (0, 0)),
            pl.BlockSpec((H, C), lambda i: (0, 0)),
        ],
        out_specs=pl.BlockSpec((EB, CP), lambda i: (i, 0)),
        out_shape=jax.ShapeDtypeStruct((E, CP), jnp.int32),
    )(edge_radial.T, edge_spherical.T, W_fc1, W_fc2)

    src, dst = pl.pallas_call(
        _split_idx_kernel,
        out_shape=[
            jax.ShapeDtypeStruct((E,), jnp.int32),
            jax.ShapeDtypeStruct((E,), jnp.int32),
        ],
    )(edge_index)

    acc = functools.partial(
        pl.kernel,
        out_type=jax.ShapeDtypeStruct((NC, N, C), jnp.float32),
        mesh=plsc.VectorSubcoreMesh(core_axis_name="c", subcore_axis_name="s",
                                    num_cores=NC, num_subcores=NS),
        scratch_types=(
            [pltpu.VMEM_SHARED((N, C), jnp.float32)]
            + [pltpu.VMEM((K,), jnp.int32)] * 6
            + [pltpu.VMEM((K, C), jnp.float32)] * 2
            + [pltpu.VMEM((K, CP), jnp.int32)] * 2
            + [pltpu.SemaphoreType.DMA] * 12
        ),
    )(_sc_kernel)(xl, w2p, src, dst)

    out = pl.pallas_call(
        _combine_kernel,
        grid=(N // NB,),
        in_specs=[
            pl.BlockSpec((NC, NB, C), lambda i: (0, i, 0)),
            pl.BlockSpec((NB, C), lambda i: (i, 0)),
        ],
        out_specs=pl.BlockSpec((NB, C), lambda i: (i, 0)),
        out_shape=jax.ShapeDtypeStruct((N, C), jnp.float32),
    )(acc, sc)
    return out


# speedup vs baseline: 1.2619x; 1.2619x over previous
import functools

import jax
import jax.numpy as jnp
from jax import lax
from jax.experimental import pallas as pl
from jax.experimental.pallas import tpu as pltpu
from jax.experimental.pallas import tpu_sc as plsc

N = 10000
E = 320000
C = 128
A = 16
R = 8
H = 8
AVG_NEIGH = 32.0

NC = 2
NS = 16
L = 16
CP = C // 2

NB = 1000
EB = 6400
K = 80
EPW = E // (NC * NS)
NBLK = EPW // K
RPT = 624
TAIL = N - NS * RPT


def _pack_bf16(v):
    return pltpu.pack_elementwise([v[:, :CP], v[:, CP:]],
                                  packed_dtype=jnp.bfloat16)


def _xl_kernel(x_ref, wl_ref, xl_ref):
    xl_ref[...] = jnp.dot(x_ref[...], wl_ref[...] * (1.0 / jnp.sqrt(float(C))),
                          preferred_element_type=jnp.float32)


def _sc_term_kernel(x_ref, na_ref, wsc_ref, sc_ref):
    xb = x_ref[...]
    acc = jnp.zeros((NB, C), dtype=jnp.float32)
    for a in range(A):
        acc = acc + jnp.dot(xb * na_ref[:, a][:, None], wsc_ref[:, a, :],
                            preferred_element_type=jnp.float32)
    sc_ref[...] = acc * (1.0 / jnp.sqrt(float(C * A)))


def _split_idx_kernel(ei_ref, src_ref, dst_ref):
    src_ref[...] = ei_ref[0]
    dst_ref[...] = ei_ref[1]


def _edge_kernel(radT_ref, sphT_ref, wf1_ref, wf2_ref, w2p_ref):
    preT = lax.dot_general(wf1_ref[...] * (1.0 / jnp.sqrt(float(R))),
                           radT_ref[...],
                           (((0,), (0,)), ((), ())),
                           preferred_element_type=jnp.float32)
    hT = (jax.nn.softplus(preT) - jnp.log(2.0)) * sphT_ref[...]
    w = lax.dot_general(hT,
                        wf2_ref[...] * (1.0 / jnp.sqrt(float(H) * AVG_NEIGH)),
                        (((0,), (0,)), ((), ())),
                        preferred_element_type=jnp.float32)
    w2p_ref[...] = _pack_bf16(w)


def _combine_kernel(acc_ref, sc_ref, out_ref):
    out_ref[...] = acc_ref[0] + acc_ref[1] + sc_ref[...]


HI_MASK = -65536


def _sc_kernel(xl_hbm, w2p_hbm, src_hbm, dst_hbm, out_hbm,
               acc, srcb0, srcb1, srcb2, dstb0, dstb1, dstb2, rows0, rows1,
               w2b0, w2b1,
               sg0, sg1, sw0, sw1, si0, si1, si2, sd0, sd1, sd2, ss0, ss1):
    cid = lax.axis_index("c")
    sid = lax.axis_index("s")
    rows = (rows0, rows1)
    w2b = (w2b0, w2b1)
    srcb = (srcb0, srcb1, srcb2)
    dstb = (dstb0, dstb1, dstb2)
    sg = (sg0, sg1)
    sw = (sw0, sw1)
    si = (si0, si1, si2)
    sd = (sd0, sd1, sd2)
    ss = (ss0, ss1)

    def zrow(i, _):
        for k in range(C // L):
            rows0[i, pl.ds(k * L, L)] = jnp.zeros((L,), jnp.float32)
        return 0
    lax.fori_loop(0, K, zrow, 0)
    r0 = sid * RPT
    for j in range(RPT // K):
        pltpu.sync_copy(rows0, acc.at[pl.ds(r0 + j * K, K)])
    rem = RPT - (RPT // K) * K
    if rem:
        pltpu.sync_copy(rows0.at[pl.ds(0, rem)],
                        acc.at[pl.ds(r0 + (RPT // K) * K, rem)])

    @pl.when(sid == NS - 1)
    def _():
        pltpu.sync_copy(rows0.at[pl.ds(0, TAIL)], acc.at[pl.ds(NS * RPT, TAIL)])
    plsc.subcore_barrier()

    base_e = (cid * NS + sid) * EPW

    def issue_idx(t, b):
        pltpu.async_copy(src_hbm.at[pl.ds(base_e + b * K, K)], srcb[t], si[t])
        pltpu.async_copy(dst_hbm.at[pl.ds(base_e + b * K, K)], dstb[t], sd[t])

    def wait_idx(t, b):
        pltpu.make_async_copy(src_hbm.at[pl.ds(base_e + b * K, K)],
                              srcb[t], si[t]).wait()
        pltpu.make_async_copy(dst_hbm.at[pl.ds(base_e + b * K, K)],
                              dstb[t], sd[t]).wait()

    def issue_gw(s, t, b):
        pltpu.async_copy(w2p_hbm.at[pl.ds(base_e + b * K, K)], w2b[s], sw[s])
        pltpu.async_copy(xl_hbm.at[srcb[t]], rows[s], sg[s])

    def step(j, b, nxt_gw, nxt_idx, wait_sc):
        s, t = j % 2, j % 3
        if nxt_idx:
            issue_idx((j + 2) % 3, b + 2)
        pltpu.make_async_copy(w2p_hbm.at[pl.ds(base_e + b * K, K)],
                              w2b[s], sw[s]).wait()
        pltpu.make_async_copy(xl_hbm.at[srcb[t]], rows[s], sg[s]).wait()
        if nxt_gw:
            s1, t1 = (j + 1) % 2, (j + 1) % 3
            if wait_sc:
                pltpu.make_async_copy(
                    rows[s1], acc.at[dstb[(j + 2) % 3]], ss[s1]).wait()
            wait_idx(t1, b + 1)
            issue_gw(s1, t1, b + 1)

        def mul(i, _):
            for g in range(C // 32):
                wv = w2b[s][i, pl.ds(g * L, L)]
                w_lo = lax.bitcast_convert_type(wv << 16, jnp.float32)
                w_hi = lax.bitcast_convert_type(wv & HI_MASK, jnp.float32)
                slo = pl.ds(g * L, L)
                shi = pl.ds(CP + g * L, L)
                rows[s][i, slo] = rows[s][i, slo] * w_lo
                rows[s][i, shi] = rows[s][i, shi] * w_hi
            return 0
        lax.fori_loop(0, K, mul, 0)
        pltpu.async_copy(rows[s], acc.at[dstb[t]], ss[s], add=True)

    issue_idx(0, 0)
    issue_idx(1, 1)
    wait_idx(0, 0)
    issue_gw(0, 0, 0)

    for b in range(6):
        step(b, b, True, True, b >= 1)

    NMAIN = 6 + (NBLK - 5 - 6) // 6 * 6

    def body(i, _):
        b0 = 6 + 6 * i
        for j in range(6):
            step(j, b0 + j, True, True, True)
        return 0
    lax.fori_loop(0, (NMAIN - 6) // 6, body, 0)
    for b in range(NMAIN, NBLK):
        step(b % 6, b, b + 1 < NBLK, b + 2 < NBLK, b + 1 < NBLK)
    jl, jl1 = (NBLK - 1) % 6, (NBLK - 2) % 6
    pltpu.make_async_copy(rows[jl % 2], acc.at[dstb[jl % 3]],
                          ss[jl % 2]).wait()
    pltpu.make_async_copy(rows[jl1 % 2], acc.at[dstb[jl1 % 3]],
                          ss[jl1 % 2]).wait()
    plsc.subcore_barrier()

    pltpu.sync_copy(acc.at[pl.ds(r0, RPT)], out_hbm.at[cid, pl.ds(r0, RPT)])

    @pl.when(sid == NS - 1)
    def _():
        pltpu.sync_copy(acc.at[pl.ds(NS * RPT, TAIL)],
                        out_hbm.at[cid, pl.ds(NS * RPT, TAIL)])


def kernel(x, node_attrs, edge_radial, edge_spherical, edge_index,
           W_lin, W_fc1, W_fc2, W_sc):
    xl = pl.pallas_call(
        _xl_kernel,
        grid=(N // NB,),
        in_specs=[
            pl.BlockSpec((NB, C), lambda i: (i, 0)),
            pl.BlockSpec((C, C), lambda i: (0, 0)),
        ],
        out_specs=pl.BlockSpec((NB, C), lambda i: (i, 0)),
        out_shape=jax.ShapeDtypeStruct((N, C), jnp.float32),
    )(x, W_lin)

    sc = pl.pallas_call(
        _sc_term_kernel,
        grid=(N // NB,),
        in_specs=[
            pl.BlockSpec((NB, C), lambda i: (i, 0)),
            pl.BlockSpec((NB, A), lambda i: (i, 0)),
            pl.BlockSpec((C, A, C), lambda i: (0, 0, 0)),
        ],
        out_specs=pl.BlockSpec((NB, C), lambda i: (i, 0)),
        out_shape=jax.ShapeDtypeStruct((N, C), jnp.float32),
    )(x, node_attrs, W_sc)

    w2p = pl.pallas_call(
        _edge_kernel,
        grid=(E // EB,),
        in_specs=[
            pl.BlockSpec((R, EB), lambda i: (0, i)),
            pl.BlockSpec((1, EB), lambda i: (0, i)),
            pl.BlockSpec((R, H), lambda i: (0, 0)),
            pl.BlockSpec((H, C), lambda i: (0, 0)),
        ],
        out_specs=pl.BlockSpec((EB, CP), lambda i: (i, 0)),
        out_shape=jax.ShapeDtypeStruct((E, CP), jnp.int32),
    )(edge_radial.T, edge_spherical.T, W_fc1, W_fc2)

    src, dst = pl.pallas_call(
        _split_idx_kernel,
        out_shape=[
            jax.ShapeDtypeStruct((E,), jnp.int32),
            jax.ShapeDtypeStruct((E,), jnp.int32),
        ],
    )(edge_index)

    acc = functools.partial(
        pl.kernel,
        out_type=jax.ShapeDtypeStruct((NC, N, C), jnp.float32),
        mesh=plsc.VectorSubcoreMesh(core_axis_name="c", subcore_axis_name="s",
                                    num_cores=NC, num_subcores=NS),
        scratch_types=(
            [pltpu.VMEM_SHARED((N, C), jnp.float32)]
            + [pltpu.VMEM((K,), jnp.int32)] * 6
            + [pltpu.VMEM((K, C), jnp.float32)] * 2
            + [pltpu.VMEM((K, CP), jnp.int32)] * 2
            + [pltpu.SemaphoreType.DMA] * 12
        ),
    )(_sc_kernel)(xl, w2p, src, dst)

    out = pl.pallas_call(
        _combine_kernel,
        grid=(N // NB,),
        in_specs=[
            pl.BlockSpec((NC, NB, C), lambda i: (0, i, 0)),
            pl.BlockSpec((NB, C), lambda i: (i, 0)),
        ],
        out_specs=pl.BlockSpec((NB, C), lambda i: (i, 0)),
        out_shape=jax.ShapeDtypeStruct((N, C), jnp.float32),
    )(acc, sc)
    return out
